# baseline (speedup 1.0000x reference)
import functools

import jax
import jax.numpy as jnp
from jax import lax
from jax.experimental import pallas as pl
from jax.experimental.pallas import tpu as pltpu
from jax.experimental.pallas import tpu_sc as plsc

WAY = 5
SHOT_SP = 5 * 512
S = 512
HW = 512 * 512
Q = 25
NQ_SP = Q * S
D = 128

NC = 2
NS = 16
NW = NC * NS
L = 16
PIX_PER_W = HW // NW
GROUPS = PIX_PER_W // L
STAGE_W = PIX_PER_W * WAY


def _predict_body(feat_ref, lab_ref, out_ref):
    shot = feat_ref[:SHOT_SP, :]
    query = feat_ref[SHOT_SP:, :]
    lab = lab_ref[...]
    onehot = (lab == lax.broadcasted_iota(jnp.int32, (SHOT_SP, WAY), 1))
    onehot = onehot.astype(jnp.float32)
    proto_sum = lax.dot_general(
        onehot, shot, (((0,), (0,)), ((), ())),
        preferred_element_type=jnp.float32,
        precision=lax.Precision.HIGHEST)
    counts = jnp.sum(onehot, axis=0)
    protos = proto_sum / jnp.maximum(counts, 1.0)[:, None]
    p2 = jnp.sum(protos * protos, axis=1)[:, None]
    qp = lax.dot_general(
        protos, query, (((1,), (1,)), ((), ())),
        preferred_element_type=jnp.float32,
        precision=lax.Precision.HIGHEST)
    z = 2.0 * qp - p2
    z = z - jnp.max(z, axis=0, keepdims=True)
    e = jnp.exp(z)
    out_ref[...] = e / jnp.sum(e, axis=0, keepdims=True)


def _compute_predict_p(feature_superpixel, label_shot):
    lab2d = label_shot.astype(jnp.int32).reshape(SHOT_SP, 1)
    return pl.pallas_call(
        _predict_body,
        out_shape=jax.ShapeDtypeStruct((WAY, NQ_SP), jnp.float32),
    )(feature_superpixel, lab2d)


def _sc_scatter_body(tbl_hbm, seg_hbm, out_hbm,
                     tbl_v0, tbl_v1, seg_v0, seg_v1, stage_v0, stage_v1,
                     sem_in, sem_out):
    tbl_v = (tbl_v0, tbl_v1)
    seg_v = (seg_v0, seg_v1)
    stage_v = (stage_v0, stage_v1)
    wid = lax.axis_index("s") * NC + lax.axis_index("c")

    def in_copies(q, b):
        cps = [pltpu.make_async_copy(
            tbl_hbm.at[:, pl.ds(q * S, S)], tbl_v[b], sem_in.at[b])]
        cps.append(pltpu.make_async_copy(
            seg_hbm.at[pl.ds((5 + q) * HW + wid * PIX_PER_W, PIX_PER_W)],
            seg_v[b], sem_in.at[b]))
        return cps

    def out_copies(q, b):
        return [pltpu.make_async_copy(
            stage_v[b].at[pl.ds(c * PIX_PER_W, PIX_PER_W)],
            out_hbm.at[pl.ds((q * WAY + c) * HW + wid * PIX_PER_W, PIX_PER_W)],
            sem_out.at[b]) for c in range(WAY)]

    def compute(b):
        cvecs = [jnp.full((L,), c, jnp.int32) for c in range(WAY)]

        @plsc.parallel_loop(0, GROUPS, unroll=16)
        def per_group(g):
            seg = seg_v[b][pl.ds(g * L, L)]
            for c in range(WAY):
                vals = plsc.load_gather(tbl_v[b], [cvecs[c], seg])
                stage_v[b][pl.ds(c * PIX_PER_W + g * L, L)] = vals

    def start(cps):
        for cp in cps:
            cp.start()

    def wait(cps):
        for cp in cps:
            cp.wait()

    start(in_copies(0, 0))
    start(in_copies(1, 1))
    wait(in_copies(0, 0))
    compute(0)
    start(out_copies(0, 0))
    start(in_copies(2, 0))
    wait(in_copies(1, 1))
    compute(1)
    start(out_copies(1, 1))
    start(in_copies(3, 1))

    def pair(i, carry):
        q0 = 2 * i
        q1 = q0 + 1
        wait(in_copies(q0, 0))
        wait(out_copies(q0 - 2, 0))
        compute(0)
        start(out_copies(q0, 0))
        start(in_copies(q0 + 2, 0))
        wait(in_copies(q1, 1))
        wait(out_copies(q1 - 2, 1))
        compute(1)
        start(out_copies(q1, 1))
        start(in_copies(q1 + 2, 1))
        return carry

    lax.fori_loop(1, 11, pair, 0)
    for q, b in ((22, 0), (23, 1), (24, 0)):
        wait(in_copies(q, b))
        wait(out_copies(q - 2, b))
        compute(b)
        start(out_copies(q, b))
        if q == 22:
            start(in_copies(24, 0))
    wait(out_copies(24, 0))
    wait(out_copies(23, 1))


def _scatter_masks(predict_flat, slic_flat):
    mesh = plsc.VectorSubcoreMesh(core_axis_name="c", subcore_axis_name="s")
    f = pl.kernel(
        _sc_scatter_body,
        mesh=mesh,
        out_type=jax.ShapeDtypeStruct((Q * HW * WAY,), jnp.float32),
        scratch_types=[
            pltpu.VMEM((WAY, S), jnp.float32),
            pltpu.VMEM((WAY, S), jnp.float32),
            pltpu.VMEM((PIX_PER_W,), jnp.int32),
            pltpu.VMEM((PIX_PER_W,), jnp.int32),
            pltpu.VMEM((STAGE_W,), jnp.float32),
            pltpu.VMEM((STAGE_W,), jnp.float32),
            pltpu.SemaphoreType.DMA((2,)),
            pltpu.SemaphoreType.DMA((2,)),
        ],
        compiler_params=pltpu.CompilerParams(needs_layout_passes=False),
    )
    return f(predict_flat, slic_flat)


def kernel(feature_superpixel, label_superpixel_shot, label_superpixel_query, slic_npy):
    predict_t = _compute_predict_p(feature_superpixel, label_superpixel_shot)
    predict_p = predict_t.T
    seg_phys = (slic_npy.astype(jnp.int32)
                .reshape(30, 64, 8, 4, 128)
                .transpose(0, 1, 3, 2, 4)
                .reshape(-1))
    out_flat = _scatter_masks(predict_t, seg_phys)
    out = (out_flat.reshape(Q, WAY, 64, 4, 8, 128)
           .transpose(0, 2, 4, 3, 5, 1)
           .reshape(Q, 512, 512, WAY))
    return out, predict_p


# speedup vs baseline: 1.0302x; 1.0302x over previous
import functools

import jax
import jax.numpy as jnp
from jax import lax
from jax.experimental import pallas as pl
from jax.experimental.pallas import tpu as pltpu
from jax.experimental.pallas import tpu_sc as plsc

WAY = 5
SHOT_SP = 5 * 512
S = 512
HW = 512 * 512
Q = 25
NQ_SP = Q * S
D = 128

NC = 2
NS = 16
NW = NC * NS
L = 16
PIX_PER_W = HW // NW
GROUPS = PIX_PER_W // L
STAGE_W = PIX_PER_W * WAY


def _predict_body(feat_ref, lab_ref, out_ref):
    shot = feat_ref[:SHOT_SP, :]
    query = feat_ref[SHOT_SP:, :]
    lab = lab_ref[...]
    onehot = (lab == lax.broadcasted_iota(jnp.int32, (SHOT_SP, WAY), 1))
    onehot = onehot.astype(jnp.float32)
    proto_sum = lax.dot_general(
        onehot, shot, (((0,), (0,)), ((), ())),
        preferred_element_type=jnp.float32,
        precision=lax.Precision.HIGHEST)
    counts = jnp.sum(onehot, axis=0)
    protos = proto_sum / jnp.maximum(counts, 1.0)[:, None]
    p2 = jnp.sum(protos * protos, axis=1)[:, None]
    qp = lax.dot_general(
        protos, query, (((1,), (1,)), ((), ())),
        preferred_element_type=jnp.float32,
        precision=lax.Precision.HIGHEST)
    z = 2.0 * qp - p2
    z = z - jnp.max(z, axis=0, keepdims=True)
    e = jnp.exp(z)
    out_ref[...] = e / jnp.sum(e, axis=0, keepdims=True)


def _compute_predict_p(feature_superpixel, label_shot):
    lab2d = label_shot.astype(jnp.int32).reshape(SHOT_SP, 1)
    return pl.pallas_call(
        _predict_body,
        out_shape=jax.ShapeDtypeStruct((WAY, NQ_SP), jnp.float32),
    )(feature_superpixel, lab2d)


def _sc_scatter_body(tbl_hbm, seg_hbm, out_hbm,
                     tbl_v0, tbl_v1, seg_v0, seg_v1, stage_v0, stage_v1,
                     sem_in, sem_out):
    tbl_v = (tbl_v0, tbl_v1)
    seg_v = (seg_v0, seg_v1)
    stage_v = (stage_v0, stage_v1)
    wid = lax.axis_index("s") * NC + lax.axis_index("c")

    def in_copies(q, b):
        cps = [pltpu.make_async_copy(
            tbl_hbm.at[:, pl.ds(q * S, S)], tbl_v[b], sem_in.at[b])]
        cps.append(pltpu.make_async_copy(
            seg_hbm.at[pl.ds((5 + q) * HW + wid * PIX_PER_W, PIX_PER_W)],
            seg_v[b], sem_in.at[b]))
        return cps

    def out_copies(q, b):
        return [pltpu.make_async_copy(
            stage_v[b].at[pl.ds(c * PIX_PER_W, PIX_PER_W)],
            out_hbm.at[pl.ds((q * WAY + c) * HW + wid * PIX_PER_W, PIX_PER_W)],
            sem_out.at[b]) for c in range(WAY)]

    def compute(b):
        cvecs = [jnp.full((L,), c, jnp.int32) for c in range(WAY)]

        @plsc.parallel_loop(0, GROUPS, unroll=4)
        def per_group(g):
            seg = seg_v[b][pl.ds(g * L, L)]
            for c in range(WAY):
                vals = plsc.load_gather(tbl_v[b], [cvecs[c], seg])
                stage_v[b][pl.ds(c * PIX_PER_W + g * L, L)] = vals

    def start(cps):
        for cp in cps:
            cp.start()

    def wait(cps):
        for cp in cps:
            cp.wait()

    start(in_copies(0, 0))
    start(in_copies(1, 1))
    wait(in_copies(0, 0))
    compute(0)
    start(out_copies(0, 0))
    start(in_copies(2, 0))
    wait(in_copies(1, 1))
    compute(1)
    start(out_copies(1, 1))
    start(in_copies(3, 1))

    def pair(i, carry):
        q0 = 2 * i
        q1 = q0 + 1
        wait(in_copies(q0, 0))
        wait(out_copies(q0 - 2, 0))
        compute(0)
        start(out_copies(q0, 0))
        start(in_copies(q0 + 2, 0))
        wait(in_copies(q1, 1))
        wait(out_copies(q1 - 2, 1))
        compute(1)
        start(out_copies(q1, 1))
        start(in_copies(q1 + 2, 1))
        return carry

    lax.fori_loop(1, 11, pair, 0)
    for q, b in ((22, 0), (23, 1), (24, 0)):
        wait(in_copies(q, b))
        wait(out_copies(q - 2, b))
        compute(b)
        start(out_copies(q, b))
        if q == 22:
            start(in_copies(24, 0))
    wait(out_copies(24, 0))
    wait(out_copies(23, 1))


def _scatter_masks(predict_flat, slic_flat):
    mesh = plsc.VectorSubcoreMesh(core_axis_name="c", subcore_axis_name="s")
    f = pl.kernel(
        _sc_scatter_body,
        mesh=mesh,
        out_type=jax.ShapeDtypeStruct((Q * HW * WAY,), jnp.float32),
        scratch_types=[
            pltpu.VMEM((WAY, S), jnp.float32),
            pltpu.VMEM((WAY, S), jnp.float32),
            pltpu.VMEM((PIX_PER_W,), jnp.int32),
            pltpu.VMEM((PIX_PER_W,), jnp.int32),
            pltpu.VMEM((STAGE_W,), jnp.float32),
            pltpu.VMEM((STAGE_W,), jnp.float32),
            pltpu.SemaphoreType.DMA((2,)),
            pltpu.SemaphoreType.DMA((2,)),
        ],
        compiler_params=pltpu.CompilerParams(needs_layout_passes=False),
    )
    return f(predict_flat, slic_flat)


def kernel(feature_superpixel, label_superpixel_shot, label_superpixel_query, slic_npy):
    predict_t = _compute_predict_p(feature_superpixel, label_superpixel_shot)
    predict_p = predict_t.T
    seg_phys = (slic_npy.astype(jnp.int32)
                .reshape(30, 64, 8, 4, 128)
                .transpose(0, 1, 3, 2, 4)
                .reshape(-1))
    out_flat = _scatter_masks(predict_t, seg_phys)
    out = (out_flat.reshape(Q, WAY, 64, 4, 8, 128)
           .transpose(0, 2, 4, 3, 5, 1)
           .reshape(Q, 512, 512, WAY))
    return out, predict_p
